# speedup vs baseline: 1.5293x; 1.5293x over previous
import functools

import jax
import jax.numpy as jnp
from jax import lax
from jax.experimental import pallas as pl
from jax.experimental.pallas import tpu as pltpu
from jax.experimental.pallas import tpu_sc as plsc

TABLE_ROWS = 8192
EMB_DIM = 1024
SEQ_LEN = 8192

NUM_CORES = 2
NUM_SUBCORES = 16
NUM_WORKERS = NUM_CORES * NUM_SUBCORES
ROWS_PER_WORKER = SEQ_LEN // NUM_WORKERS
CHUNK = 32
NCHUNKS = ROWS_PER_WORKER // CHUNK
NBUF = 2

_MESH = plsc.VectorSubcoreMesh(core_axis_name="c", subcore_axis_name="s")


@functools.partial(
    pl.kernel,
    mesh=_MESH,
    out_type=jax.ShapeDtypeStruct((SEQ_LEN, EMB_DIM), jnp.float32),
    scratch_types=[
        pltpu.VMEM((NCHUNKS, CHUNK), jnp.int32),
        pltpu.VMEM((CHUNK, EMB_DIM), jnp.float32),
        pltpu.VMEM((CHUNK, EMB_DIM), jnp.float32),
        pltpu.SemaphoreType.DMA,
        pltpu.SemaphoreType.DMA,
    ],
)
def _sc_gather(idx_hbm, table_hbm, out_hbm, idx_v, buf0, buf1, sem0, sem1):
    wid = lax.axis_index("s") * NUM_CORES + lax.axis_index("c")
    base = wid * ROWS_PER_WORKER
    pltpu.sync_copy(idx_hbm.at[wid], idx_v)
    bufs = (buf0, buf1)
    sems = (sem0, sem1)
    handles = [None] * NCHUNKS
    handles[0] = pltpu.async_copy(table_hbm.at[idx_v.at[0]], bufs[0], sems[0])
    for c in range(NCHUNKS):
        if c + 1 < NCHUNKS:
            nb = (c + 1) % NBUF
            handles[c + 1] = pltpu.async_copy(
                table_hbm.at[idx_v.at[c + 1]], bufs[nb], sems[nb]
            )
        handles[c].wait()
        pltpu.sync_copy(bufs[c % NBUF],
                        out_hbm.at[pl.ds(base + c * CHUNK, CHUNK)])


def kernel(seq_len, pos_emb_weight):
    offset = jnp.asarray(seq_len, jnp.int32) - SEQ_LEN
    ids = jnp.arange(SEQ_LEN, dtype=jnp.int32) + offset
    ids = jnp.clip(ids, 0, TABLE_ROWS - 1)
    idx = ids.reshape(NUM_WORKERS, NCHUNKS, CHUNK)
    return _sc_gather(idx, pos_emb_weight)


# speedup vs baseline: 1.5720x; 1.0279x over previous
import functools

import jax
import jax.numpy as jnp
from jax import lax
from jax.experimental import pallas as pl
from jax.experimental.pallas import tpu as pltpu
from jax.experimental.pallas import tpu_sc as plsc

TABLE_ROWS = 8192
EMB_DIM = 1024
SEQ_LEN = 8192

NUM_CORES = 2
NUM_SUBCORES = 16
NUM_WORKERS = NUM_CORES * NUM_SUBCORES
ROWS_PER_WORKER = SEQ_LEN // NUM_WORKERS
CHUNK = 32
NCHUNKS = ROWS_PER_WORKER // CHUNK
NBUF = 3

_MESH = plsc.VectorSubcoreMesh(core_axis_name="c", subcore_axis_name="s")


@functools.partial(
    pl.kernel,
    mesh=_MESH,
    out_type=jax.ShapeDtypeStruct((SEQ_LEN, EMB_DIM), jnp.float32),
    scratch_types=[
        pltpu.VMEM((NCHUNKS, CHUNK), jnp.int32),
        pltpu.VMEM((CHUNK, EMB_DIM), jnp.float32),
        pltpu.VMEM((CHUNK, EMB_DIM), jnp.float32),
        pltpu.VMEM((CHUNK, EMB_DIM), jnp.float32),
        pltpu.SemaphoreType.DMA,
        pltpu.SemaphoreType.DMA,
        pltpu.SemaphoreType.DMA,
        pltpu.SemaphoreType.DMA,
        pltpu.SemaphoreType.DMA,
        pltpu.SemaphoreType.DMA,
    ],
)
def _sc_gather(idx_hbm, table_hbm, out_hbm, idx_v,
               buf0, buf1, buf2, g0, g1, g2, s0, s1, s2):
    wid = lax.axis_index("s") * NUM_CORES + lax.axis_index("c")
    base = wid * ROWS_PER_WORKER
    pltpu.sync_copy(idx_hbm.at[wid], idx_v)
    bufs = (buf0, buf1, buf2)
    gsem = (g0, g1, g2)
    ssem = (s0, s1, s2)
    gh = [None] * NCHUNKS
    sh = [None] * NCHUNKS
    for c in range(min(NBUF - 1, NCHUNKS)):
        gh[c] = pltpu.async_copy(table_hbm.at[idx_v.at[c]],
                                 bufs[c % NBUF], gsem[c % NBUF])
    for c in range(NCHUNKS):
        nxt = c + NBUF - 1
        if nxt < NCHUNKS:
            if nxt - NBUF >= 0:
                sh[nxt - NBUF].wait()
            gh[nxt] = pltpu.async_copy(table_hbm.at[idx_v.at[nxt]],
                                       bufs[nxt % NBUF], gsem[nxt % NBUF])
        gh[c].wait()
        sh[c] = pltpu.async_copy(bufs[c % NBUF],
                                 out_hbm.at[pl.ds(base + c * CHUNK, CHUNK)],
                                 ssem[c % NBUF])
    for c in range(max(0, NCHUNKS - NBUF), NCHUNKS):
        sh[c].wait()


def kernel(seq_len, pos_emb_weight):
    offset = jnp.asarray(seq_len, jnp.int32) - SEQ_LEN
    ids = jnp.arange(SEQ_LEN, dtype=jnp.int32) + offset
    ids = jnp.clip(ids, 0, TABLE_ROWS - 1)
    idx = ids.reshape(NUM_WORKERS, NCHUNKS, CHUNK)
    return _sc_gather(idx, pos_emb_weight)


# speedup vs baseline: 1.5796x; 1.0048x over previous
import functools

import jax
import jax.numpy as jnp
from jax import lax
from jax.experimental import pallas as pl
from jax.experimental.pallas import tpu as pltpu
from jax.experimental.pallas import tpu_sc as plsc

TABLE_ROWS = 8192
EMB_DIM = 1024
SEQ_LEN = 8192

NUM_CORES = 2
NUM_SUBCORES = 16
NUM_WORKERS = NUM_CORES * NUM_SUBCORES
ROWS_PER_WORKER = SEQ_LEN // NUM_WORKERS
CHUNK = 16
NCHUNKS = ROWS_PER_WORKER // CHUNK
NBUF = 6

_MESH = plsc.VectorSubcoreMesh(core_axis_name="c", subcore_axis_name="s")


@functools.partial(
    pl.kernel,
    mesh=_MESH,
    out_type=jax.ShapeDtypeStruct((SEQ_LEN, EMB_DIM), jnp.float32),
    scratch_types=(
        [pltpu.VMEM((NCHUNKS, CHUNK), jnp.int32)]
        + [pltpu.VMEM((CHUNK, EMB_DIM), jnp.float32)] * NBUF
        + [pltpu.SemaphoreType.DMA] * (2 * NBUF)
    ),
)
def _sc_gather(idx_hbm, table_hbm, out_hbm, idx_v, *scratch):
    wid = lax.axis_index("s") * NUM_CORES + lax.axis_index("c")
    base = wid * ROWS_PER_WORKER
    pltpu.sync_copy(idx_hbm.at[wid], idx_v)
    bufs = scratch[:NBUF]
    gsem = scratch[NBUF:2 * NBUF]
    ssem = scratch[2 * NBUF:]
    gh = [None] * NCHUNKS
    sh = [None] * NCHUNKS
    for c in range(min(NBUF - 1, NCHUNKS)):
        gh[c] = pltpu.async_copy(table_hbm.at[idx_v.at[c]],
                                 bufs[c % NBUF], gsem[c % NBUF])
    for c in range(NCHUNKS):
        nxt = c + NBUF - 1
        if nxt < NCHUNKS:
            if nxt - NBUF >= 0:
                sh[nxt - NBUF].wait()
            gh[nxt] = pltpu.async_copy(table_hbm.at[idx_v.at[nxt]],
                                       bufs[nxt % NBUF], gsem[nxt % NBUF])
        gh[c].wait()
        sh[c] = pltpu.async_copy(bufs[c % NBUF],
                                 out_hbm.at[pl.ds(base + c * CHUNK, CHUNK)],
                                 ssem[c % NBUF])
    for c in range(max(0, NCHUNKS - NBUF), NCHUNKS):
        sh[c].wait()


def kernel(seq_len, pos_emb_weight):
    offset = jnp.asarray(seq_len, jnp.int32) - SEQ_LEN
    ids = jnp.arange(SEQ_LEN, dtype=jnp.int32) + offset
    ids = jnp.clip(ids, 0, TABLE_ROWS - 1)
    idx = ids.reshape(NUM_WORKERS, NCHUNKS, CHUNK)
    return _sc_gather(idx, pos_emb_weight)
